# baseline (speedup 1.0000x reference)
import functools

import jax
import jax.numpy as jnp
import numpy as np
from jax import lax
from jax.experimental import pallas as pl
from jax.experimental.pallas import tpu as pltpu
from jax.experimental.pallas import tpu_sc as plsc

_N = 32768
_K = 655
_L = 16
_NV = _N // _L
_SIGN = np.uint32(0x80000000)


def _sc_body(x_hbm, out_hbm, xv, kv, hist, outv):
    cid = lax.axis_index("c")
    sid = lax.axis_index("s")

    @pl.when(jnp.logical_and(cid == 0, sid == 0))
    def _():
        pltpu.sync_copy(x_hbm, xv)
        lanes = lax.iota(jnp.int32, _L)
        zeros_f = jnp.zeros((_L,), jnp.float32)
        zeros_i = jnp.zeros((_L,), jnp.int32)
        ones_i = jnp.ones((_L,), jnp.int32)

        def keygen(i, acc):
            v = xv[pl.ds(i * _L, _L)]
            u = lax.bitcast_convert_type(v, jnp.uint32)
            key = jnp.where((u >> 31) == np.uint32(0), u | _SIGN, ~u)
            kv[pl.ds(i * _L, _L)] = key
            return acc + v * v

        acc = lax.fori_loop(0, _NV, keygen, zeros_f)
        sumsq = jnp.sum(acc)

        def zero_hist(i, c):
            hist[pl.ds(i * _L, _L)] = zeros_i
            return c

        lax.fori_loop(0, 256, zero_hist, 0)

        prefix = np.uint32(0)
        kr = np.int32(_K)
        for shift in (24, 16, 8, 0):

            def hist_pass(i, c, shift=shift, prefix=prefix):
                k = kv[pl.ds(i * _L, _L)]
                bucket = ((k >> shift) & np.uint32(0xFF)).astype(jnp.int32)
                idx = bucket * _L + lanes
                if shift == 24:
                    plsc.addupdate_scatter(hist, [idx], ones_i)
                else:
                    m = ((k ^ prefix) >> (shift + 8)) == np.uint32(0)
                    plsc.addupdate_scatter(hist, [idx], ones_i, mask=m)
                return c

            lax.fori_loop(0, _NV, hist_pass, 0)

            def scan(j, carry, kr=kr):
                cum, sel, above = carry
                b = 255 - j
                row = hist[pl.ds(b * _L, _L)]
                hist[pl.ds(b * _L, _L)] = zeros_i
                ncum = cum + jnp.sum(row)
                found = jnp.logical_and(cum < kr, ncum >= kr)
                sel = jnp.where(found, b, sel)
                above = jnp.where(found, cum, above)
                return ncum, sel, above

            _, sel, above = lax.fori_loop(
                0, 256, scan, (np.int32(0), np.int32(0), np.int32(0)))
            prefix = prefix | (sel.astype(jnp.uint32) << shift)
            kr = kr - above

        t = prefix

        def final(i, carry):
            sacc, cacc = carry
            k = kv[pl.ds(i * _L, _L)]
            m = k > t
            u = jnp.where(k >= _SIGN, k ^ _SIGN, ~k)
            v = lax.bitcast_convert_type(u, jnp.float32)
            return (sacc + jnp.where(m, v, zeros_f),
                    cacc + jnp.where(m, ones_i, zeros_i))

        sacc, cacc = lax.fori_loop(0, _NV, final, (zeros_f, zeros_i))
        sum_gt = jnp.sum(sacc)
        cnt_gt = jnp.sum(cacc)

        tvec = jnp.full((_L,), t)
        ut = jnp.where(tvec >= _SIGN, tvec ^ _SIGN, ~tvec)
        val_t = lax.bitcast_convert_type(ut, jnp.float32)
        top = sum_gt + (np.int32(_K) - cnt_gt).astype(jnp.float32) * val_t

        svec = jnp.full((_L,), sumsq)
        i0 = np.uint32(0x5F3759DF) - (lax.bitcast_convert_type(svec, jnp.uint32) >> 1)
        y = lax.bitcast_convert_type(i0, jnp.float32)
        for _ in range(3):
            y = y * (1.5 - 0.5 * svec * y * y)
        norm = jnp.maximum(svec * y, jnp.full((_L,), np.float32(1e-12)))
        outv[...] = jnp.where(svec > 0, top / norm, zeros_f)
        pltpu.sync_copy(outv, out_hbm)


_topk_sum_sc = functools.partial(
    pl.kernel,
    out_type=jax.ShapeDtypeStruct((_L,), jnp.float32),
    mesh=plsc.VectorSubcoreMesh(
        core_axis_name="c", subcore_axis_name="s",
        num_cores=2, num_subcores=16),
    compiler_params=pltpu.CompilerParams(needs_layout_passes=False),
    scratch_types=[
        pltpu.VMEM((_N,), jnp.float32),
        pltpu.VMEM((_N,), jnp.uint32),
        pltpu.VMEM((256 * _L,), jnp.int32),
        pltpu.VMEM((_L,), jnp.float32),
    ],
)(_sc_body)


def kernel(glb_feature, aux):
    x = jnp.reshape(glb_feature, (_N,))
    return _topk_sum_sc(x)[0]


# speedup vs baseline: 3.7152x; 3.7152x over previous
import functools

import jax
import jax.numpy as jnp
import numpy as np
from jax import lax
from jax.experimental import pallas as pl
from jax.experimental.pallas import tpu as pltpu
from jax.experimental.pallas import tpu_sc as plsc

_N = 32768
_K = 655
_L = 16
_NT = 16
_C = _N // _NT
_CV = _C // _L
_SIGN = np.uint32(0x80000000)


def _splat_i(x):
    return jnp.full((_L,), x, dtype=jnp.int32)


def _splat_f(x):
    return jnp.full((_L,), x, dtype=jnp.float32)


def _suffix(v):
    r = lax.rev(v, dimensions=(0,))
    return lax.rev(plsc.cumsum(r), dimensions=(0,))


def _sc_body(x_hbm, out_hbm, xv, kv, plh, psh, rh, rs, ghl, gsl, lacc,
             outv, zv, zvf, gh0, gh1, gh2, gh3, gs, gacc):
    cid = lax.axis_index("c")
    sid = lax.axis_index("s")
    lanes = lax.iota(jnp.int32, _L)
    zeros_f = jnp.zeros((_L,), jnp.float32)
    zeros_i = jnp.zeros((_L,), jnp.int32)
    ones_i = jnp.ones((_L,), jnp.int32)
    ghs = (gh0, gh1, gh2, gh3)

    pltpu.sync_copy(x_hbm.at[pl.ds(sid * _C, _C)], xv)
    for g in range(_L):
        zv[g] = zeros_i
        zvf[g] = zeros_f
        lacc[g] = zeros_f

    @pl.when(sid == 0)
    def _():
        for gh in ghs:
            pltpu.sync_copy(zv, gh)
        pltpu.sync_copy(zvf, gs)
        pltpu.sync_copy(zvf, gacc)

    def zero_lh(i, c):
        plh[pl.ds(i * _L, _L)] = zeros_i
        psh[pl.ds(i * _L, _L)] = zeros_f
        return c

    lax.fori_loop(0, 256, zero_lh, 0)

    def r0(i, acc):
        v = xv[pl.ds(i * _L, _L)]
        u = lax.bitcast_convert_type(v, jnp.uint32)
        key = jnp.where((u >> 31) == np.uint32(0), u | _SIGN, ~u)
        kv[pl.ds(i * _L, _L)] = key
        bhi = (key >> 28).astype(jnp.int32)
        blo = ((key >> 24) & np.uint32(0xF)).astype(jnp.int32)
        idx = lanes * 256 + blo * _L + bhi
        plsc.addupdate_scatter(plh, [idx], ones_i)
        plsc.addupdate_scatter(psh, [idx], v)
        return acc + v * v

    sumsq_v = lax.fori_loop(0, _CV, r0, zeros_f)
    plsc.subcore_barrier()

    def merge_and_scan(rnd, kr_v, with_sums):
        def red(g, c):
            ci = zeros_i
            cf = zeros_f
            for l in range(_L):
                ci = ci + plh[pl.ds(l * 256 + g * _L, _L)]
                plh[pl.ds(l * 256 + g * _L, _L)] = zeros_i
                if with_sums:
                    cf = cf + psh[pl.ds(l * 256 + g * _L, _L)]
                    psh[pl.ds(l * 256 + g * _L, _L)] = zeros_f
            rh[g] = ci
            if with_sums:
                rs[g] = cf
            return c

        lax.fori_loop(0, _L, red, 0)
        pltpu.sync_copy(rh, ghs[rnd].at[lanes], add=True)
        if with_sums:
            pltpu.sync_copy(rs, gs.at[lanes], add=True)
        plsc.subcore_barrier()
        pltpu.sync_copy(ghs[rnd], ghl)
        if with_sums:
            pltpu.sync_copy(gs, gsl)

        ltot = zeros_i
        stot = zeros_f
        for g in range(_L):
            ltot = ltot + ghl[g]
            if with_sums:
                stot = stot + gsl[g]
        sl = _suffix(ltot)
        l_sel = plsc.all_reduce_population_count(sl >= kr_v) - 1
        above1 = jnp.sum(jnp.where(lanes > l_sel, ltot, zeros_i))
        minor = plsc.load_gather(ghl, [lanes, l_sel])
        sm = _suffix(minor) + _splat_i(above1)
        c_sel = plsc.all_reduce_population_count(sm >= kr_v) - 1
        above2 = jnp.sum(jnp.where(lanes > c_sel, minor, zeros_i)) + above1
        if with_sums:
            sminor = plsc.load_gather(gsl, [lanes, l_sel])
            sum_hi = (jnp.sum(jnp.where(lanes > l_sel, stot, zeros_f)) +
                      jnp.sum(jnp.where(lanes > c_sel, sminor, zeros_f)))
        else:
            sum_hi = jnp.float32(0)
        return l_sel * _L + c_sel, above2, sum_hi

    kr_v = _splat_i(_K)
    sel, cnt_hi, sum_hi = merge_and_scan(0, kr_v, True)
    kr_v = kr_v - _splat_i(cnt_hi)
    prefix_v = sel.astype(jnp.uint32) << 24

    for rnd, shift in ((1, 16), (2, 8), (3, 0)):

        def rr(i, c, shift=shift, prefix_v=prefix_v):
            k = kv[pl.ds(i * _L, _L)]
            m = ((k ^ prefix_v) >> (shift + 8)) == np.uint32(0)
            bhi = ((k >> (shift + 4)) & np.uint32(0xF)).astype(jnp.int32)
            blo = ((k >> shift) & np.uint32(0xF)).astype(jnp.int32)
            idx = lanes * 256 + blo * _L + bhi
            plsc.addupdate_scatter(plh, [idx], ones_i, mask=m)
            return c

        lax.fori_loop(0, _CV, rr, 0)
        sel, above, _ = merge_and_scan(rnd, kr_v, False)
        kr_v = kr_v - _splat_i(above)
        prefix_v = prefix_v | (sel.astype(jnp.uint32) << shift)

    t_v = prefix_v

    def fin(i, carry):
        s, cn = carry
        k = kv[pl.ds(i * _L, _L)]
        m = k > t_v
        u = jnp.where(k >= _SIGN, k ^ _SIGN, ~k)
        v = lax.bitcast_convert_type(u, jnp.float32)
        return (s + jnp.where(m, v, zeros_f),
                cn + jnp.where(m, jnp.full((_L,), jnp.float32(1)), zeros_f))

    s_lo, c_lo = lax.fori_loop(0, _CV, fin, (zeros_f, zeros_f))
    lacc[0] = sumsq_v
    lacc[1] = s_lo
    lacc[2] = c_lo
    pltpu.sync_copy(lacc, gacc.at[lanes], add=True)
    plsc.subcore_barrier()
    pltpu.sync_copy(gacc, gsl)

    sumsq = jnp.sum(gsl[0])
    sum_gt = jnp.sum(gsl[1]) + sum_hi
    cnt_gt = jnp.sum(gsl[2]) + cnt_hi.astype(jnp.float32)

    ut = jnp.where(t_v >= _SIGN, t_v ^ _SIGN, ~t_v)
    val_t = lax.bitcast_convert_type(ut, jnp.float32)
    top = _splat_f(sum_gt) + (_splat_f(np.float32(_K)) - _splat_f(cnt_gt)) * val_t

    svec = _splat_f(sumsq)
    i0 = np.uint32(0x5F3759DF) - (lax.bitcast_convert_type(svec, jnp.uint32) >> 1)
    y = lax.bitcast_convert_type(i0, jnp.float32)
    for _ in range(3):
        y = y * (1.5 - 0.5 * svec * y * y)
    norm = jnp.maximum(svec * y, _splat_f(np.float32(1e-12)))
    outv[...] = jnp.where(svec > 0, top / norm, zeros_f)

    @pl.when(jnp.logical_and(cid == 0, sid == 0))
    def _():
        pltpu.sync_copy(outv, out_hbm)


_topk_sum_sc = functools.partial(
    pl.kernel,
    out_type=jax.ShapeDtypeStruct((_L,), jnp.float32),
    mesh=plsc.VectorSubcoreMesh(
        core_axis_name="c", subcore_axis_name="s",
        num_cores=2, num_subcores=16),
    compiler_params=pltpu.CompilerParams(
        needs_layout_passes=False, use_tc_tiling_on_sc=False),
    scratch_types=[
        pltpu.VMEM((_C,), jnp.float32),
        pltpu.VMEM((_C,), jnp.uint32),
        pltpu.VMEM((16 * 256,), jnp.int32),
        pltpu.VMEM((16 * 256,), jnp.float32),
        pltpu.VMEM((_L, _L), jnp.int32),
        pltpu.VMEM((_L, _L), jnp.float32),
        pltpu.VMEM((_L, _L), jnp.int32),
        pltpu.VMEM((_L, _L), jnp.float32),
        pltpu.VMEM((_L, _L), jnp.float32),
        pltpu.VMEM((_L,), jnp.float32),
        pltpu.VMEM((_L, _L), jnp.int32),
        pltpu.VMEM((_L, _L), jnp.float32),
        pltpu.VMEM_SHARED((_L, _L), jnp.int32),
        pltpu.VMEM_SHARED((_L, _L), jnp.int32),
        pltpu.VMEM_SHARED((_L, _L), jnp.int32),
        pltpu.VMEM_SHARED((_L, _L), jnp.int32),
        pltpu.VMEM_SHARED((_L, _L), jnp.float32),
        pltpu.VMEM_SHARED((_L, _L), jnp.float32),
    ],
)(_sc_body)


def kernel(glb_feature, aux):
    x = jnp.reshape(glb_feature, (_N,))
    return _topk_sum_sc(x)[0]


# speedup vs baseline: 4.4714x; 1.2036x over previous
import functools

import jax
import jax.numpy as jnp
import numpy as np
from jax import lax
from jax.experimental import pallas as pl
from jax.experimental.pallas import tpu as pltpu
from jax.experimental.pallas import tpu_sc as plsc

_N = 32768
_K = 655
_L = 16
_NT = 16
_C = _N // _NT
_CV = _C // _L
_SIGN = np.uint32(0x80000000)


def _splat_i(x):
    return jnp.full((_L,), x, dtype=jnp.int32)


def _splat_f(x):
    return jnp.full((_L,), x, dtype=jnp.float32)


def _suffix(v):
    r = lax.rev(v, dimensions=(0,))
    return lax.rev(plsc.cumsum(r), dimensions=(0,))


def _sc_body(x_hbm, out_hbm, xv, kv, cand, rcs, ghl, lacc, outv, zvf,
             gcs, gh1, gh2, gh3, gacc):
    sid = lax.axis_index("s")
    lanes = lax.iota(jnp.int32, _L)
    zeros_f = jnp.zeros((_L,), jnp.float32)
    ones_f = jnp.ones((_L,), jnp.float32)

    pltpu.sync_copy(x_hbm.at[pl.ds(sid * _C, _C)], xv)
    for g in range(2 * _L):
        zvf[g] = zeros_f
    for g in range(_L):
        lacc[g] = zeros_f

    @pl.when(sid == 0)
    def _():
        pltpu.sync_copy(zvf, gcs)
        pltpu.sync_copy(zvf.at[pl.ds(0, _L)], gh1)
        pltpu.sync_copy(zvf.at[pl.ds(0, _L)], gh2)
        pltpu.sync_copy(zvf.at[pl.ds(0, _L)], gh3)
        pltpu.sync_copy(zvf.at[pl.ds(0, _L)], gacc)

    def zero_rcs(n):
        def z(i, c):
            rcs[i] = zeros_f
            return c
        lax.fori_loop(0, n, z, 0)

    zero_rcs(32)

    def r0(i, acc):
        v = xv[pl.ds(i * _L, _L)]
        u = lax.bitcast_convert_type(v, jnp.uint32)
        key = jnp.where((u >> 31) == np.uint32(0), u | _SIGN, ~u)
        kv[pl.ds(i * _L, _L)] = key
        bhi = (key >> 28).astype(jnp.int32)
        blo = ((key >> 24) & np.uint32(0xF)).astype(jnp.int32)
        plsc.addupdate_scatter(rcs, [blo, bhi], ones_f)
        plsc.addupdate_scatter(rcs, [blo + _L, bhi], v)
        return acc + v * v

    sumsq_v = lax.fori_loop(0, _CV, r0, zeros_f)
    plsc.subcore_barrier()
    pltpu.sync_copy(rcs.at[pl.ds(0, _L)], gcs.at[lanes], add=True)
    pltpu.sync_copy(rcs.at[pl.ds(_L, _L)], gcs.at[lanes + _L], add=True)
    plsc.subcore_barrier()
    pltpu.sync_copy(gcs, ghl)

    def scan(kr_v, rows, srows):
        ltot = zeros_f
        for g in range(_L):
            ltot = ltot + rows[g]
        sl = _suffix(ltot)
        l_sel = plsc.all_reduce_population_count(sl >= kr_v) - 1
        above1 = jnp.sum(jnp.where(lanes > l_sel, ltot, zeros_f))
        minor = plsc.load_gather(ghl, [lanes, l_sel])
        sm = _suffix(minor) + _splat_f(above1)
        c_sel = plsc.all_reduce_population_count(sm >= kr_v) - 1
        above2 = jnp.sum(jnp.where(lanes > c_sel, minor, zeros_f)) + above1
        if srows is not None:
            stot = zeros_f
            for g in range(_L):
                stot = stot + srows[g]
            sminor = plsc.load_gather(ghl, [lanes + _L, l_sel])
            sum_hi = (jnp.sum(jnp.where(lanes > l_sel, stot, zeros_f)) +
                      jnp.sum(jnp.where(lanes > c_sel, sminor, zeros_f)))
        else:
            sum_hi = None
        return l_sel * _L + c_sel, _splat_f(above2), sum_hi

    kr_v = _splat_f(np.float32(_K))
    sel, cnt_hi_v, sum_hi = scan(
        kr_v, [ghl[g] for g in range(_L)], [ghl[g + _L] for g in range(_L)])
    kr_v = kr_v - cnt_hi_v
    prefix_v = sel.astype(jnp.uint32) << 24

    zero_rcs(16)

    def comp(i, off_v):
        k = kv[pl.ds(i * _L, _L)]
        m = (k >> 24) == (prefix_v >> 24)
        pc = plsc.cumsum(m.astype(jnp.int32))
        plsc.store_scatter(cand, [off_v + pc - 1],
                           lax.bitcast_convert_type(k, jnp.int32), mask=m)
        bhi = ((k >> 20) & np.uint32(0xF)).astype(jnp.int32)
        blo = ((k >> 16) & np.uint32(0xF)).astype(jnp.int32)
        plsc.addupdate_scatter(rcs, [blo, bhi], ones_f, mask=m)
        return off_v + plsc.all_reduce_population_count(m)

    nc_v = lax.fori_loop(0, _CV, comp, _splat_i(0))
    nvec = (jnp.max(nc_v) + _L - 1) // _L

    for rnd, (gh, shift) in enumerate(((gh1, 16), (gh2, 8), (gh3, 0))):
        if rnd > 0:
            zero_rcs(16)

            def rr(i, c, shift=shift, prefix_v=prefix_v):
                k = lax.bitcast_convert_type(cand[pl.ds(i * _L, _L)],
                                             jnp.uint32)
                valid = (i * _L + lanes) < nc_v
                m = (((k ^ prefix_v) >> (shift + 8)) == np.uint32(0)) & valid
                bhi = ((k >> (shift + 4)) & np.uint32(0xF)).astype(jnp.int32)
                blo = ((k >> shift) & np.uint32(0xF)).astype(jnp.int32)
                plsc.addupdate_scatter(rcs, [blo, bhi], ones_f, mask=m)
                return c

            lax.fori_loop(0, nvec, rr, 0)
        pltpu.sync_copy(rcs.at[pl.ds(0, _L)], gh.at[lanes], add=True)
        plsc.subcore_barrier()
        pltpu.sync_copy(gh, ghl.at[pl.ds(0, _L)])
        sel, above_v, _ = scan(kr_v, [ghl[g] for g in range(_L)], None)
        kr_v = kr_v - above_v
        prefix_v = prefix_v | (sel.astype(jnp.uint32) << shift)

    t_v = prefix_v

    def fin(i, carry):
        s, cn = carry
        k = lax.bitcast_convert_type(cand[pl.ds(i * _L, _L)], jnp.uint32)
        valid = (i * _L + lanes) < nc_v
        m = (k > t_v) & valid
        u = jnp.where(k >= _SIGN, k ^ _SIGN, ~k)
        v = lax.bitcast_convert_type(u, jnp.float32)
        return s + jnp.where(m, v, zeros_f), cn + jnp.where(m, ones_f, zeros_f)

    s_lo, c_lo = lax.fori_loop(0, nvec, fin, (zeros_f, zeros_f))
    lacc[0] = sumsq_v
    lacc[1] = s_lo
    lacc[2] = c_lo
    pltpu.sync_copy(lacc, gacc.at[lanes], add=True)
    plsc.subcore_barrier()
    pltpu.sync_copy(gacc, ghl.at[pl.ds(0, _L)])

    sumsq = jnp.sum(ghl[0])
    sum_gt = jnp.sum(ghl[1]) + sum_hi
    cnt_gt = jnp.sum(ghl[2]) + jnp.max(cnt_hi_v)

    ut = jnp.where(t_v >= _SIGN, t_v ^ _SIGN, ~t_v)
    val_t = lax.bitcast_convert_type(ut, jnp.float32)
    top = _splat_f(sum_gt) + (_splat_f(np.float32(_K)) - _splat_f(cnt_gt)) * val_t

    svec = _splat_f(sumsq)
    i0 = np.uint32(0x5F3759DF) - (lax.bitcast_convert_type(svec, jnp.uint32) >> 1)
    y = lax.bitcast_convert_type(i0, jnp.float32)
    for _ in range(3):
        y = y * (1.5 - 0.5 * svec * y * y)
    norm = jnp.maximum(svec * y, _splat_f(np.float32(1e-12)))
    outv[...] = jnp.where(svec > 0, top / norm, zeros_f)

    @pl.when(sid == 0)
    def _():
        pltpu.sync_copy(outv, out_hbm)


_topk_sum_sc = functools.partial(
    pl.kernel,
    out_type=jax.ShapeDtypeStruct((_L,), jnp.float32),
    mesh=plsc.VectorSubcoreMesh(
        core_axis_name="c", subcore_axis_name="s",
        num_cores=1, num_subcores=16),
    compiler_params=pltpu.CompilerParams(
        needs_layout_passes=False, use_tc_tiling_on_sc=False),
    scratch_types=[
        pltpu.VMEM((_C,), jnp.float32),
        pltpu.VMEM((_C,), jnp.uint32),
        pltpu.VMEM((_C,), jnp.int32),
        pltpu.VMEM((2 * _L, _L), jnp.float32),
        pltpu.VMEM((2 * _L, _L), jnp.float32),
        pltpu.VMEM((_L, _L), jnp.float32),
        pltpu.VMEM((_L,), jnp.float32),
        pltpu.VMEM((2 * _L, _L), jnp.float32),
        pltpu.VMEM_SHARED((2 * _L, _L), jnp.float32),
        pltpu.VMEM_SHARED((_L, _L), jnp.float32),
        pltpu.VMEM_SHARED((_L, _L), jnp.float32),
        pltpu.VMEM_SHARED((_L, _L), jnp.float32),
        pltpu.VMEM_SHARED((_L, _L), jnp.float32),
    ],
)(_sc_body)


def kernel(glb_feature, aux):
    x = jnp.reshape(glb_feature, (_N,))
    return _topk_sum_sc(x)[0]
